# baseline (speedup 1.0000x reference)
import jax
import jax.numpy as jnp
from jax import lax
from jax.experimental import pallas as pl
from jax.experimental.pallas import tpu as pltpu
from jax.experimental.pallas import tpu_sc as plsc

_N = 10000
_D = 256
_DH = 128
_E = 160000
_NC = 2
_NS = 16

_K = 64
_KD = 128
_EPAD = 163840
_EPT = _EPAD // _NS
_NCH = _EPT // _K
_NCHD = _EPAD // (_NC * _NS) // _KD
_NROW = _EPAD // _K
_NROWD = _EPAD // _KD

_NP = 10240
_RPT = _NP // _NS
_RC = 64
_NRC = _RPT // _RC

_MESH = plsc.VectorSubcoreMesh(core_axis_name="c", subcore_axis_name="s")


def _deg_body(dst2d_hbm, ones_hbm, zeros_hbm, out_hbm,
              acc, didx_v, ones_v, tmp_v, s0):
    c = lax.axis_index("c")
    s = lax.axis_index("s")
    w = s * _NC + c
    pltpu.sync_copy(dst2d_hbm.at[pl.ds(w * _NCHD, _NCHD)], didx_v)
    pltpu.sync_copy(ones_hbm, ones_v)
    pltpu.sync_copy(zeros_hbm, tmp_v)
    for i in range(_NRC):
        pltpu.sync_copy(tmp_v, acc.at[pl.ds(s * _RPT + i * _RC, _RC)])
    plsc.subcore_barrier()

    def body(k, carry):
        pltpu.async_copy(ones_v, acc.at[didx_v.at[k]], s0, add=True)
        pltpu.make_async_copy(ones_hbm, ones_v, s0).wait()
        return carry

    lax.fori_loop(0, _NCHD, body, 0)
    plsc.subcore_barrier()
    for i in range(_NRC):
        r0 = s * _RPT + i * _RC
        pltpu.sync_copy(acc.at[pl.ds(r0, _RC)], tmp_v)
        pltpu.sync_copy(tmp_v, out_hbm.at[pl.ds(c * _NP + r0, _RC)])


_deg_call = pl.kernel(
    _deg_body,
    out_type=jax.ShapeDtypeStruct((_NC * _NP, _DH), jnp.float32),
    mesh=_MESH,
    scratch_types=[
        pltpu.VMEM_SHARED((_NP, _DH), jnp.float32),
        pltpu.VMEM((_NCHD, _KD), jnp.int32),
        pltpu.VMEM((_KD, _DH), jnp.float32),
        pltpu.VMEM((_RC, _DH), jnp.float32),
        pltpu.SemaphoreType.DMA,
    ],
)


def _agg_body(h_hbm, idx2_hbm, zeros_hbm, out_hbm,
              acc, idx_v, ra_v, rb_v, rc_v, g0, g1, g2, s0, s1, s2, i0):
    c = lax.axis_index("c")
    s = lax.axis_index("s")
    rbase = s * _NCH
    rows = (ra_v, rb_v, rc_v)
    gs = (g0, g1, g2)
    ss = (s0, s1, s2)

    pltpu.async_copy(idx2_hbm.at[c, pl.ds(rbase, _NCH)], idx_v, i0)
    pltpu.sync_copy(zeros_hbm, ra_v)
    for i in range(_NRC):
        pltpu.sync_copy(ra_v, acc.at[pl.ds(s * _RPT + i * _RC, _RC)])
    pltpu.make_async_copy(idx2_hbm.at[c, pl.ds(0, _NCH)], idx_v, i0).wait()
    plsc.subcore_barrier()

    def gfire(k, b):
        pltpu.async_copy(h_hbm.at[idx_v.at[k, pl.ds(0, _K)]], rows[b], gs[b])

    def gwait(b):
        pltpu.make_async_copy(h_hbm.at[pl.ds(0, _K)], rows[b], gs[b]).wait()

    def sfire(k, b):
        pltpu.async_copy(rows[b], acc.at[idx_v.at[k, pl.ds(_K, _K)]], ss[b],
                         add=True)

    def swait(b):
        pltpu.make_async_copy(h_hbm.at[pl.ds(0, _K)], rows[b], ss[b]).wait()

    gfire(0, 0)
    gwait(0)
    sfire(0, 0)
    gfire(1, 1)
    gwait(1)
    sfire(1, 1)
    gfire(2, 2)

    def body(j, carry):
        for t in range(3):
            k = 3 * j + 2 + t
            b = (2 + t) % 3
            gwait(b)
            swait(t)
            sfire(k, b)
            gfire(k + 1, t)
        return carry

    lax.fori_loop(0, (_NCH - 4) // 3, body, 0)
    gwait((_NCH - 2) % 3)
    swait((_NCH - 4) % 3)
    sfire(_NCH - 2, (_NCH - 2) % 3)
    gfire(_NCH - 1, (_NCH - 4) % 3)
    gwait((_NCH - 4) % 3)
    swait((_NCH - 3) % 3)
    sfire(_NCH - 1, (_NCH - 4) % 3)
    swait((_NCH - 2) % 3)
    swait((_NCH - 4) % 3)
    plsc.subcore_barrier()
    for i in range(_NRC):
        r0 = s * _RPT + i * _RC
        pltpu.sync_copy(acc.at[pl.ds(r0, _RC)], ra_v)
        pltpu.sync_copy(ra_v, out_hbm.at[pl.ds(c * _NP + r0, _RC)])


_agg_call = pl.kernel(
    _agg_body,
    out_type=jax.ShapeDtypeStruct((_NC * _NP, _DH), jnp.float32),
    mesh=_MESH,
    scratch_types=[
        pltpu.VMEM_SHARED((_NP, _DH), jnp.float32),
        pltpu.VMEM((_NCH, 2 * _K), jnp.int32),
        pltpu.VMEM((_K, _DH), jnp.float32),
        pltpu.VMEM((_K, _DH), jnp.float32),
        pltpu.VMEM((_K, _DH), jnp.float32),
        pltpu.SemaphoreType.DMA,
        pltpu.SemaphoreType.DMA,
        pltpu.SemaphoreType.DMA,
        pltpu.SemaphoreType.DMA,
        pltpu.SemaphoreType.DMA,
        pltpu.SemaphoreType.DMA,
        pltpu.SemaphoreType.DMA,
    ],
)


_B = 1000
_G = _N // _B


def _dinv_from(dg_ref):
    dsum = dg_ref[0, :, 0:1] + dg_ref[1, :, 0:1] + 1.0
    return lax.rsqrt(dsum)


def _lin1_body(x_ref, w_ref, dg_ref, out_ref):
    h = jnp.dot(x_ref[...], w_ref[...], preferred_element_type=jnp.float32)
    ht = h * _dinv_from(dg_ref)
    out_ref[0] = ht[:, :_DH]
    out_ref[1] = ht[:, _DH:]


def _lin2_body(s_ref, h_ref, dg_ref, w_ref, b1_ref, out_ref):
    dinv = _dinv_from(dg_ref)
    xa = jnp.maximum(dinv * (s_ref[0] + h_ref[0]) + b1_ref[:, :_DH], 0.0)
    xb = jnp.maximum(dinv * (s_ref[1] + h_ref[1]) + b1_ref[:, _DH:], 0.0)
    x2 = jnp.concatenate([xa, xb], axis=1)
    h2 = jnp.dot(x2, w_ref[...], preferred_element_type=jnp.float32)
    ht = h2 * dinv
    out_ref[0] = ht[:, :_DH]
    out_ref[1] = ht[:, _DH:]


def _fin_body(s_ref, h_ref, dg_ref, b2_ref, out_ref):
    dinv = _dinv_from(dg_ref)
    oa = dinv * (s_ref[0] + h_ref[0]) + b2_ref[:, :_DH]
    ob = dinv * (s_ref[1] + h_ref[1]) + b2_ref[:, _DH:]
    out_ref[...] = jnp.concatenate([oa, ob], axis=1)


_half_spec = pl.BlockSpec((2, _B, _DH), lambda i: (0, i, 0))
_w_spec = pl.BlockSpec((_D, _D), lambda i: (0, 0))
_b_spec = pl.BlockSpec((1, _D), lambda i: (0, 0))
_half_out = jax.ShapeDtypeStruct((2, _N, _DH), jnp.float32)

_lin1_call = pl.pallas_call(
    _lin1_body,
    grid=(_G,),
    in_specs=[pl.BlockSpec((_B, _D), lambda i: (i, 0)), _w_spec, _half_spec],
    out_specs=_half_spec,
    out_shape=_half_out,
)

_lin2_call = pl.pallas_call(
    _lin2_body,
    grid=(_G,),
    in_specs=[_half_spec, _half_spec, _half_spec, _w_spec, _b_spec],
    out_specs=_half_spec,
    out_shape=_half_out,
)

_fin_call = pl.pallas_call(
    _fin_body,
    grid=(_G,),
    in_specs=[_half_spec, _half_spec, _half_spec, _b_spec],
    out_specs=pl.BlockSpec((_B, _D), lambda i: (i, 0)),
    out_shape=jax.ShapeDtypeStruct((_N, _D), jnp.float32),
)


def kernel(x, edge_index, W1, b1, W2, b2):
    assert x.shape == (_N, _D) and edge_index.shape == (2, _E)
    src = edge_index[0].astype(jnp.int32)
    dst = edge_index[1].astype(jnp.int32)
    pad = _EPAD - _E
    src_p = jnp.concatenate([src, jnp.zeros((pad,), jnp.int32)])
    dst_p = jnp.concatenate([dst, jnp.full((pad,), _N, jnp.int32)])
    srcv = src_p.reshape(_NROW, _K)
    dstv = dst_p.reshape(_NROW, _K)
    dstd = dst_p.reshape(_NROWD, _KD)
    idx2 = jnp.stack([jnp.concatenate([srcv, dstv], axis=1),
                      jnp.concatenate([srcv + _N, dstv], axis=1)])
    zeros_h = jnp.zeros((_RC, _DH), jnp.float32)
    ones_d = jnp.ones((_KD, _DH), jnp.float32)

    deg2 = _deg_call(dstd, ones_d, zeros_h).reshape(_NC, _NP, _DH)
    h1 = _lin1_call(x, W1, deg2)
    s1 = _agg_call(h1.reshape(_NC * _N, _DH), idx2, zeros_h)
    h2 = _lin2_call(s1.reshape(_NC, _NP, _DH), h1, deg2, W2,
                    b1.reshape(1, _D))
    s2 = _agg_call(h2.reshape(_NC * _N, _DH), idx2, zeros_h)
    return _fin_call(s2.reshape(_NC, _NP, _DH), h2, deg2, b2.reshape(1, _D))


# speedup vs baseline: 1.0182x; 1.0182x over previous
import jax
import jax.numpy as jnp
from jax import lax
from jax.experimental import pallas as pl
from jax.experimental.pallas import tpu as pltpu
from jax.experimental.pallas import tpu_sc as plsc

_N = 10000
_D = 256
_DH = 128
_E = 160000
_NC = 2
_NS = 16

_K = 128
_KD = 128
_EPAD = 163840
_EPT = _EPAD // _NS
_NCH = _EPT // _K
_NCHH = _NCH // 2
_NCHD = _EPAD // (_NC * _NS) // _KD
_NROW = _EPAD // _K
_NROWD = _EPAD // _KD

_NP = 10240
_RPT = _NP // _NS
_RC = 128
_NRC = _RPT // _RC

_MESH = plsc.VectorSubcoreMesh(core_axis_name="c", subcore_axis_name="s")


def _deg_body(dst2d_hbm, ones_hbm, zeros_hbm, out_hbm,
              acc, didx_v, ones_v, tmp_v, s0):
    c = lax.axis_index("c")
    s = lax.axis_index("s")
    w = s * _NC + c
    pltpu.sync_copy(dst2d_hbm.at[pl.ds(w * _NCHD, _NCHD)], didx_v)
    pltpu.sync_copy(ones_hbm, ones_v)
    pltpu.sync_copy(zeros_hbm, tmp_v)
    for i in range(_NRC):
        pltpu.sync_copy(tmp_v, acc.at[pl.ds(s * _RPT + i * _RC, _RC)])
    plsc.subcore_barrier()

    def body(k, carry):
        pltpu.async_copy(ones_v, acc.at[didx_v.at[k]], s0, add=True)
        pltpu.make_async_copy(ones_hbm, ones_v, s0).wait()
        return carry

    lax.fori_loop(0, _NCHD, body, 0)
    plsc.subcore_barrier()
    for i in range(_NRC):
        r0 = s * _RPT + i * _RC
        pltpu.sync_copy(acc.at[pl.ds(r0, _RC)], tmp_v)
        pltpu.sync_copy(tmp_v, out_hbm.at[pl.ds(c * _NP + r0, _RC)])


_deg_call = pl.kernel(
    _deg_body,
    out_type=jax.ShapeDtypeStruct((_NC * _NP, _DH), jnp.float32),
    mesh=_MESH,
    scratch_types=[
        pltpu.VMEM_SHARED((_NP, _DH), jnp.float32),
        pltpu.VMEM((_NCHD, _KD), jnp.int32),
        pltpu.VMEM((_KD, _DH), jnp.float32),
        pltpu.VMEM((_RC, _DH), jnp.float32),
        pltpu.SemaphoreType.DMA,
    ],
)


def _agg_body(h_hbm, idx2_hbm, zeros_hbm, out_hbm,
              acc, idx_v, ra_v, rb_v, g0, g1, s0, s1, i0):
    c = lax.axis_index("c")
    s = lax.axis_index("s")
    rbase = s * _NCH
    rows = (ra_v, rb_v)
    gs = (g0, g1)
    ss = (s0, s1)

    pltpu.async_copy(idx2_hbm.at[c, pl.ds(rbase, _NCHH)], idx_v, i0)
    pltpu.sync_copy(zeros_hbm, ra_v)
    for i in range(_NRC):
        b = i % 2
        if i >= 2:
            pltpu.make_async_copy(ra_v, acc.at[pl.ds(s * _RPT, _RC)],
                                  ss[b]).wait()
        pltpu.async_copy(ra_v, acc.at[pl.ds(s * _RPT + i * _RC, _RC)], ss[b])
    for b in range(2):
        pltpu.make_async_copy(ra_v, acc.at[pl.ds(s * _RPT, _RC)], ss[b]).wait()
    pltpu.make_async_copy(idx2_hbm.at[c, pl.ds(0, _NCHH)], idx_v, i0).wait()
    plsc.subcore_barrier()

    def gfire(k, b):
        pltpu.async_copy(h_hbm.at[idx_v.at[k, pl.ds(0, _K)]], rows[b], gs[b])

    def gwait(b):
        pltpu.make_async_copy(h_hbm.at[pl.ds(0, _K)], rows[b], gs[b]).wait()

    def sfire(k, b):
        pltpu.async_copy(rows[b], acc.at[idx_v.at[k, pl.ds(_K, _K)]], ss[b],
                         add=True)

    def swait(b):
        pltpu.make_async_copy(h_hbm.at[pl.ds(0, _K)], rows[b], ss[b]).wait()

    def run_half():
        gfire(0, 0)
        gwait(0)
        sfire(0, 0)
        gfire(1, 1)

        def body(j, carry):
            for t in range(2):
                k = 2 * j + 1 + t
                b = (1 + t) % 2
                gwait(b)
                swait(t)
                sfire(k, b)
                gfire(k + 1, t)
            return carry

        lax.fori_loop(0, (_NCHH - 2) // 2, body, 0)
        gwait(1)
        swait(0)
        sfire(_NCHH - 1, 1)
        swait(1)

    run_half()
    pltpu.sync_copy(idx2_hbm.at[c, pl.ds(rbase + _NCHH, _NCHH)], idx_v)
    run_half()
    plsc.subcore_barrier()
    for i in range(_NRC):
        b = i % 2
        r0 = s * _RPT + i * _RC
        if i >= 2:
            pltpu.make_async_copy(rows[b], out_hbm.at[pl.ds(c * _NP, _RC)],
                                  gs[b]).wait()
        pltpu.sync_copy(acc.at[pl.ds(r0, _RC)], rows[b])
        pltpu.async_copy(rows[b], out_hbm.at[pl.ds(c * _NP + r0, _RC)], gs[b])
    for b in range(2):
        pltpu.make_async_copy(rows[b], out_hbm.at[pl.ds(c * _NP, _RC)],
                              gs[b]).wait()


_agg_call = pl.kernel(
    _agg_body,
    out_type=jax.ShapeDtypeStruct((_NC * _NP, _DH), jnp.float32),
    mesh=_MESH,
    scratch_types=[
        pltpu.VMEM_SHARED((_NP, _DH), jnp.float32),
        pltpu.VMEM((_NCHH, 2 * _K), jnp.int32),
        pltpu.VMEM((_K, _DH), jnp.float32),
        pltpu.VMEM((_K, _DH), jnp.float32),
        pltpu.SemaphoreType.DMA,
        pltpu.SemaphoreType.DMA,
        pltpu.SemaphoreType.DMA,
        pltpu.SemaphoreType.DMA,
        pltpu.SemaphoreType.DMA,
    ],
)


_B = 1000
_G = _N // _B


def _dinv_from(dg_ref):
    dsum = dg_ref[0, :, 0:1] + dg_ref[1, :, 0:1] + 1.0
    return lax.rsqrt(dsum)


def _lin1_body(x_ref, w_ref, dg_ref, out_ref):
    h = jnp.dot(x_ref[...], w_ref[...], preferred_element_type=jnp.float32)
    ht = h * _dinv_from(dg_ref)
    out_ref[0] = ht[:, :_DH]
    out_ref[1] = ht[:, _DH:]


def _lin2_body(s_ref, h_ref, dg_ref, w_ref, b1_ref, out_ref):
    dinv = _dinv_from(dg_ref)
    xa = jnp.maximum(dinv * (s_ref[0] + h_ref[0]) + b1_ref[:, :_DH], 0.0)
    xb = jnp.maximum(dinv * (s_ref[1] + h_ref[1]) + b1_ref[:, _DH:], 0.0)
    x2 = jnp.concatenate([xa, xb], axis=1)
    h2 = jnp.dot(x2, w_ref[...], preferred_element_type=jnp.float32)
    ht = h2 * dinv
    out_ref[0] = ht[:, :_DH]
    out_ref[1] = ht[:, _DH:]


def _fin_body(s_ref, h_ref, dg_ref, b2_ref, out_ref):
    dinv = _dinv_from(dg_ref)
    oa = dinv * (s_ref[0] + h_ref[0]) + b2_ref[:, :_DH]
    ob = dinv * (s_ref[1] + h_ref[1]) + b2_ref[:, _DH:]
    out_ref[...] = jnp.concatenate([oa, ob], axis=1)


_half_spec = pl.BlockSpec((2, _B, _DH), lambda i: (0, i, 0))
_w_spec = pl.BlockSpec((_D, _D), lambda i: (0, 0))
_b_spec = pl.BlockSpec((1, _D), lambda i: (0, 0))
_half_out = jax.ShapeDtypeStruct((2, _N, _DH), jnp.float32)

_lin1_call = pl.pallas_call(
    _lin1_body,
    grid=(_G,),
    in_specs=[pl.BlockSpec((_B, _D), lambda i: (i, 0)), _w_spec, _half_spec],
    out_specs=_half_spec,
    out_shape=_half_out,
)

_lin2_call = pl.pallas_call(
    _lin2_body,
    grid=(_G,),
    in_specs=[_half_spec, _half_spec, _half_spec, _w_spec, _b_spec],
    out_specs=_half_spec,
    out_shape=_half_out,
)

_fin_call = pl.pallas_call(
    _fin_body,
    grid=(_G,),
    in_specs=[_half_spec, _half_spec, _half_spec, _b_spec],
    out_specs=pl.BlockSpec((_B, _D), lambda i: (i, 0)),
    out_shape=jax.ShapeDtypeStruct((_N, _D), jnp.float32),
)


def kernel(x, edge_index, W1, b1, W2, b2):
    assert x.shape == (_N, _D) and edge_index.shape == (2, _E)
    src = edge_index[0].astype(jnp.int32)
    dst = edge_index[1].astype(jnp.int32)
    pad = _EPAD - _E
    src_p = jnp.concatenate([src, jnp.zeros((pad,), jnp.int32)])
    dst_p = jnp.concatenate([dst, jnp.full((pad,), _N, jnp.int32)])
    srcv = src_p.reshape(_NROW, _K)
    dstv = dst_p.reshape(_NROW, _K)
    dstd = dst_p.reshape(_NROWD, _KD)
    idx2 = jnp.stack([jnp.concatenate([srcv, dstv], axis=1),
                      jnp.concatenate([srcv + _N, dstv], axis=1)])
    zeros_h = jnp.zeros((_RC, _DH), jnp.float32)
    ones_d = jnp.ones((_KD, _DH), jnp.float32)

    deg2 = _deg_call(dstd, ones_d, zeros_h).reshape(_NC, _NP, _DH)
    h1 = _lin1_call(x, W1, deg2)
    s1 = _agg_call(h1.reshape(_NC * _N, _DH), idx2, zeros_h)
    h2 = _lin2_call(s1.reshape(_NC, _NP, _DH), h1, deg2, W2,
                    b1.reshape(1, _D))
    s2 = _agg_call(h2.reshape(_NC * _N, _DH), idx2, zeros_h)
    return _fin_call(s2.reshape(_NC, _NP, _DH), h2, deg2, b2.reshape(1, _D))
